# baseline (speedup 1.0000x reference)
import functools

import jax
import jax.numpy as jnp
from jax import lax
from jax.experimental import pallas as pl
from jax.experimental.pallas import tpu as pltpu
from jax.experimental.pallas import tpu_sc as plsc

B, C, N, K, COUT = 2, 128, 10000, 16, 128
NB = 1000
NW = 32
NPW = (B * N) // NW
NPWP = 632
G = 8
EPG = G * K
PSTEPS = NPWP // G
LANES = 16


def _tc_tables_body(x_ref, w_ref, b_ref, y1_ref, y2_ref):
    xb = x_ref[0]
    w2 = w_ref[:, C:]
    a = w_ref[:, :C] - w2
    dn = (((1,), (1,)), ((), ()))
    y1_ref[...] = lax.dot_general(xb, a, dn, preferred_element_type=jnp.float32)
    y2_ref[...] = (lax.dot_general(xb, w2, dn, preferred_element_type=jnp.float32)
                   + b_ref[...])


_tc_tables = pl.pallas_call(
    _tc_tables_body,
    grid=(B, N // NB),
    in_specs=[
        pl.BlockSpec((1, NB, C), lambda b, j: (b, j, 0)),
        pl.BlockSpec((COUT, 2 * C), lambda b, j: (0, 0)),
        pl.BlockSpec((1, COUT), lambda b, j: (0, 0)),
    ],
    out_specs=[
        pl.BlockSpec((NB, COUT), lambda b, j: (b * (N // NB) + j, 0)),
        pl.BlockSpec((NB, COUT), lambda b, j: (b * (N // NB) + j, 0)),
    ],
    out_shape=[
        jax.ShapeDtypeStruct((B * N, COUT), jnp.float32),
        jax.ShapeDtypeStruct((B * N, COUT), jnp.float32),
    ],
)


@functools.partial(
    pl.kernel,
    mesh=plsc.VectorSubcoreMesh(core_axis_name="c", subcore_axis_name="s"),
    out_type=jax.ShapeDtypeStruct((NW * PSTEPS, G, COUT), jnp.float32),
    scratch_types=(
        [pltpu.VMEM((NPWP * K,), jnp.int32)] * 2
        + [pltpu.VMEM((EPG, COUT), jnp.float32) for _ in range(2)]
        + [pltpu.VMEM((G, COUT), jnp.float32) for _ in range(2)]
        + [pltpu.SemaphoreType.DMA] * 6
    ),
)
def _sc_edge_max(y1_hbm, y2_hbm, i1_hbm, i0_hbm, out_hbm, *refs):
    i1v, i0v = refs[0], refs[1]
    rs = refs[2:4]
    outs = refs[4:6]
    s1 = refs[6:8]
    s2 = refs[8:10]
    so = refs[10:12]

    wid = lax.axis_index("s") * 2 + lax.axis_index("c")
    edge_base = wid * NPWP * K
    step_base = wid * PSTEPS

    def issue1(s, b):
        pltpu.async_copy(y1_hbm.at[i1v.at[pl.ds(s * EPG, EPG)]], rs[b], s1[b])

    def wait1(b):
        pltpu.make_async_copy(y1_hbm.at[i1v.at[pl.ds(0, EPG)]], rs[b], s1[b]).wait()

    def issue2(s, b):
        pltpu.async_copy(y2_hbm.at[i0v.at[pl.ds(s * EPG, EPG)]], rs[b], s2[b],
                         add=True)

    def wait2(b):
        pltpu.make_async_copy(y2_hbm.at[i0v.at[pl.ds(0, EPG)]], rs[b], s2[b]).wait()

    def drain_store(b):
        pltpu.make_async_copy(outs[b], out_hbm.at[0], so[b]).wait()

    def compute_store(s, b):
        r, outv = rs[b], outs[b]
        for g in range(G):
            for cb in range(COUT // LANES):
                co = cb * LANES
                vals = [r[g * K + k, pl.ds(co, LANES)] for k in range(K)]
                while len(vals) > 1:
                    vals = [jnp.maximum(vals[i], vals[i + 1])
                            for i in range(0, len(vals), 2)]
                outv[g, pl.ds(co, LANES)] = jnp.maximum(vals[0], 0.0)
        pltpu.async_copy(outv, out_hbm.at[step_base + s], so[b])

    pltpu.sync_copy(i1_hbm.at[pl.ds(edge_base, NPWP * K)], i1v)
    pltpu.sync_copy(i0_hbm.at[pl.ds(edge_base, NPWP * K)], i0v)

    issue1(0, 0)
    issue1(1, 1)
    wait1(0)
    issue2(0, 0)

    def half(s, b, t):
        wait1(b ^ 1)
        issue2(s + 1, b ^ 1)
        wait2(b)

        @pl.when(t > 0)
        def _():
            drain_store(b)
        compute_store(s, b)

        @pl.when(s + 2 < PSTEPS)
        def _():
            issue1(s + 2, b)

    def body(t, carry):
        half(2 * t, 0, t)
        half(2 * t + 1, 1, t)
        return carry

    lax.fori_loop(0, (PSTEPS - 1) // 2, body, 0)

    wait2(0)
    drain_store(0)
    compute_store(PSTEPS - 1, 0)
    drain_store(0)
    drain_store(1)


def kernel(x, edge_index, W, b):
    xT = jnp.transpose(x[..., 0], (0, 2, 1))
    y1, y2 = _tc_tables(xT, W, b.reshape(1, COUT))
    offs = (jnp.arange(B, dtype=jnp.int32) * N).reshape(B, 1, 1)
    i1f = (edge_index[1] + offs).reshape(NW, NPW * K)
    i0f = (edge_index[0] + offs).reshape(NW, NPW * K)
    pad = ((0, 0), (0, (NPWP - NPW) * K))
    i1p = jnp.pad(i1f, pad).reshape(-1)
    i0p = jnp.pad(i0f, pad).reshape(-1)
    out = _sc_edge_max(y1, y2, i1p, i0p)
    out = out.reshape(NW, NPWP, COUT)[:, :NPW]
    out = out.reshape(B, N, COUT)
    return out.transpose(0, 2, 1)[..., None]


# speedup vs baseline: 1.6494x; 1.6494x over previous
import functools

import jax
import jax.numpy as jnp
from jax import lax
from jax.experimental import pallas as pl
from jax.experimental.pallas import tpu as pltpu
from jax.experimental.pallas import tpu_sc as plsc

B, C, N, K, COUT = 2, 128, 10000, 16, 128
NB = 1000
NW = 32
NPW = (B * N) // NW
G = 5
EPG = G * K
STEPS = NPW // G
NBUF = 4
LANES = 16


def _tc_tables_body(x_ref, w_ref, b_ref, y1_ref, y2_ref):
    xb = x_ref[0]
    w2 = w_ref[:, C:]
    a = w_ref[:, :C] - w2
    dn = (((1,), (1,)), ((), ()))
    y1_ref[...] = lax.dot_general(xb, a, dn, preferred_element_type=jnp.float32)
    y2_ref[...] = (lax.dot_general(xb, w2, dn, preferred_element_type=jnp.float32)
                   + b_ref[...])


_tc_tables = pl.pallas_call(
    _tc_tables_body,
    grid=(B, N // NB),
    in_specs=[
        pl.BlockSpec((1, NB, C), lambda b, j: (b, j, 0)),
        pl.BlockSpec((COUT, 2 * C), lambda b, j: (0, 0)),
        pl.BlockSpec((1, COUT), lambda b, j: (0, 0)),
    ],
    out_specs=[
        pl.BlockSpec((NB, COUT), lambda b, j: (b * (N // NB) + j, 0)),
        pl.BlockSpec((NB, COUT), lambda b, j: (b * (N // NB) + j, 0)),
    ],
    out_shape=[
        jax.ShapeDtypeStruct((B * N, COUT), jnp.float32),
        jax.ShapeDtypeStruct((B * N, COUT), jnp.float32),
    ],
)


@functools.partial(
    pl.kernel,
    mesh=plsc.VectorSubcoreMesh(core_axis_name="c", subcore_axis_name="s"),
    out_type=jax.ShapeDtypeStruct((B * N * COUT,), jnp.float32),
    scratch_types=(
        [pltpu.VMEM((NPW * K,), jnp.int32)] * 2
        + [pltpu.VMEM((EPG, COUT), jnp.float32) for _ in range(NBUF)]
        + [pltpu.VMEM((G * COUT,), jnp.float32) for _ in range(NBUF)]
        + [pltpu.SemaphoreType.DMA] * (3 * NBUF)
    ),
)
def _sc_edge_max(y1_hbm, y2_hbm, i1_hbm, i0_hbm, out_hbm, *refs):
    i1v, i0v = refs[0], refs[1]
    rs = refs[2:2 + NBUF]
    outs = refs[2 + NBUF:2 + 2 * NBUF]
    sems = refs[2 + 2 * NBUF:]
    s1 = sems[0:NBUF]
    s2 = sems[NBUF:2 * NBUF]
    so = sems[2 * NBUF:3 * NBUF]

    wid = lax.axis_index("s") * 2 + lax.axis_index("c")
    node_base = wid * NPW
    edge_base = node_base * K

    def issue1(s, b):
        pltpu.async_copy(y1_hbm.at[i1v.at[pl.ds(s * EPG, EPG)]], rs[b], s1[b])

    def wait1(b):
        pltpu.make_async_copy(y1_hbm.at[i1v.at[pl.ds(0, EPG)]], rs[b], s1[b]).wait()

    def issue2(s, b):
        pltpu.async_copy(y2_hbm.at[i0v.at[pl.ds(s * EPG, EPG)]], rs[b], s2[b],
                         add=True)

    def wait2(b):
        pltpu.make_async_copy(y2_hbm.at[i0v.at[pl.ds(0, EPG)]], rs[b], s2[b]).wait()

    def drain_store(b):
        pltpu.make_async_copy(outs[b], out_hbm.at[pl.ds(0, G * COUT)], so[b]).wait()

    def compute_store(s, b):
        r, outv = rs[b], outs[b]
        for g in range(G):
            for cb in range(COUT // LANES):
                co = cb * LANES
                vals = [r[g * K + k, pl.ds(co, LANES)] for k in range(K)]
                while len(vals) > 1:
                    vals = [jnp.maximum(vals[i], vals[i + 1])
                            for i in range(0, len(vals), 2)]
                outv[pl.ds(g * COUT + co, LANES)] = jnp.maximum(vals[0], 0.0)
        pltpu.async_copy(
            outv, out_hbm.at[pl.ds((node_base + s * G) * COUT, G * COUT)], so[b])

    pltpu.sync_copy(i1_hbm.at[pl.ds(edge_base, NPW * K)], i1v)
    pltpu.sync_copy(i0_hbm.at[pl.ds(edge_base, NPW * K)], i0v)

    for b in range(NBUF):
        issue1(b, b)
    for b in range(2):
        wait1(b)
        issue2(b, b)

    def half(s, b, t):
        b2 = (b + 2) % NBUF

        @pl.when(s + 2 < STEPS)
        def _():
            wait1(b2)
            issue2(s + 2, b2)
        wait2(b)

        @pl.when(t > 0)
        def _():
            drain_store(b)
        compute_store(s, b)

        @pl.when(s + NBUF < STEPS)
        def _():
            issue1(s + NBUF, b)

    def body(t, carry):
        for b in range(NBUF):
            half(NBUF * t + b, b, t)
        return carry

    main_iters = STEPS // NBUF
    lax.fori_loop(0, main_iters, body, 0)

    for s in range(NBUF * main_iters, STEPS):
        b = s % NBUF
        wait2(b)
        drain_store(b)
        compute_store(s, b)
    for b in range(NBUF):
        drain_store(b)


def kernel(x, edge_index, W, b):
    xT = jnp.transpose(x[..., 0], (0, 2, 1))
    y1, y2 = _tc_tables(xT, W, b.reshape(1, COUT))
    offs = (jnp.arange(B, dtype=jnp.int32) * N).reshape(B, 1, 1)
    i1f = (edge_index[1] + offs).reshape(-1)
    i0f = (edge_index[0] + offs).reshape(-1)
    out = _sc_edge_max(y1, y2, i1f, i0f)
    return out.reshape(B, N, COUT).transpose(0, 2, 1)[..., None]


# speedup vs baseline: 1.6584x; 1.0054x over previous
import functools

import jax
import jax.numpy as jnp
from jax import lax
from jax.experimental import pallas as pl
from jax.experimental.pallas import tpu as pltpu
from jax.experimental.pallas import tpu_sc as plsc

B, C, N, K, COUT = 2, 128, 10000, 16, 128
NB = 1000
NW = 32
NPW = (B * N) // NW
G = 5
EPG = G * K
STEPS = NPW // G
NBUF = 4
LANES = 16


def _tc_tables_body(x_ref, w_ref, b_ref, y1_ref, y2_ref):
    xb = x_ref[0]
    w2 = w_ref[:, C:]
    a = w_ref[:, :C] - w2
    dn = (((0,), (1,)), ((), ()))
    y1_ref[...] = lax.dot_general(xb, a, dn, preferred_element_type=jnp.float32)
    y2_ref[...] = (lax.dot_general(xb, w2, dn, preferred_element_type=jnp.float32)
                   + b_ref[...])


_tc_tables = pl.pallas_call(
    _tc_tables_body,
    grid=(B,),
    in_specs=[
        pl.BlockSpec((1, C, N), lambda b: (b, 0, 0)),
        pl.BlockSpec((COUT, 2 * C), lambda b: (0, 0)),
        pl.BlockSpec((1, COUT), lambda b: (0, 0)),
    ],
    out_specs=[
        pl.BlockSpec((N, COUT), lambda b: (b, 0)),
        pl.BlockSpec((N, COUT), lambda b: (b, 0)),
    ],
    out_shape=[
        jax.ShapeDtypeStruct((B * N, COUT), jnp.float32),
        jax.ShapeDtypeStruct((B * N, COUT), jnp.float32),
    ],
)


@functools.partial(
    pl.kernel,
    mesh=plsc.VectorSubcoreMesh(core_axis_name="c", subcore_axis_name="s"),
    out_type=jax.ShapeDtypeStruct((B * N * COUT,), jnp.float32),
    scratch_types=(
        [pltpu.VMEM((NPW * K,), jnp.int32)] * 2
        + [pltpu.VMEM((EPG, COUT), jnp.float32) for _ in range(NBUF)]
        + [pltpu.VMEM((G * COUT,), jnp.float32) for _ in range(NBUF)]
        + [pltpu.SemaphoreType.DMA] * (3 * NBUF)
    ),
)
def _sc_edge_max(y1_hbm, y2_hbm, i1_hbm, i0_hbm, out_hbm, *refs):
    i1v, i0v = refs[0], refs[1]
    rs = refs[2:2 + NBUF]
    outs = refs[2 + NBUF:2 + 2 * NBUF]
    sems = refs[2 + 2 * NBUF:]
    s1 = sems[0:NBUF]
    s2 = sems[NBUF:2 * NBUF]
    so = sems[2 * NBUF:3 * NBUF]

    wid = lax.axis_index("s") * 2 + lax.axis_index("c")
    node_base = wid * NPW
    edge_base = node_base * K

    def issue1(s, b):
        pltpu.async_copy(y1_hbm.at[i1v.at[pl.ds(s * EPG, EPG)]], rs[b], s1[b])

    def wait1(b):
        pltpu.make_async_copy(y1_hbm.at[i1v.at[pl.ds(0, EPG)]], rs[b], s1[b]).wait()

    def issue2(s, b):
        pltpu.async_copy(y2_hbm.at[i0v.at[pl.ds(s * EPG, EPG)]], rs[b], s2[b],
                         add=True)

    def wait2(b):
        pltpu.make_async_copy(y2_hbm.at[i0v.at[pl.ds(0, EPG)]], rs[b], s2[b]).wait()

    def drain_store(b):
        pltpu.make_async_copy(outs[b], out_hbm.at[pl.ds(0, G * COUT)], so[b]).wait()

    def compute_store(s, b):
        r, outv = rs[b], outs[b]
        for g in range(G):
            for cb in range(COUT // LANES):
                co = cb * LANES
                vals = [r[g * K + k, pl.ds(co, LANES)] for k in range(K)]
                while len(vals) > 1:
                    vals = [jnp.maximum(vals[i], vals[i + 1])
                            for i in range(0, len(vals), 2)]
                outv[pl.ds(g * COUT + co, LANES)] = jnp.maximum(vals[0], 0.0)
        pltpu.async_copy(
            outv, out_hbm.at[pl.ds((node_base + s * G) * COUT, G * COUT)], so[b])

    pltpu.sync_copy(i1_hbm.at[pl.ds(edge_base, NPW * K)], i1v)
    pltpu.sync_copy(i0_hbm.at[pl.ds(edge_base, NPW * K)], i0v)

    for b in range(NBUF):
        issue1(b, b)
    for b in range(2):
        wait1(b)
        issue2(b, b)

    def half(s, b, t):
        b2 = (b + 2) % NBUF

        @pl.when(s + 2 < STEPS)
        def _():
            wait1(b2)
            issue2(s + 2, b2)
        wait2(b)

        @pl.when(t > 0)
        def _():
            drain_store(b)
        compute_store(s, b)

        @pl.when(s + NBUF < STEPS)
        def _():
            issue1(s + NBUF, b)

    def body(t, carry):
        for b in range(NBUF):
            half(NBUF * t + b, b, t)
        return carry

    main_iters = STEPS // NBUF
    lax.fori_loop(0, main_iters, body, 0)

    for s in range(NBUF * main_iters, STEPS):
        b = s % NBUF
        wait2(b)
        drain_store(b)
        compute_store(s, b)
    for b in range(NBUF):
        drain_store(b)


def kernel(x, edge_index, W, b):
    y1, y2 = _tc_tables(x[..., 0], W, b.reshape(1, COUT))
    offs = (jnp.arange(B, dtype=jnp.int32) * N).reshape(B, 1, 1)
    i1f = (edge_index[1] + offs).reshape(-1)
    i0f = (edge_index[0] + offs).reshape(-1)
    out = _sc_edge_max(y1, y2, i1f, i0f)
    return out.reshape(B, N, COUT).transpose(0, 2, 1)[..., None]


# speedup vs baseline: 2.6155x; 1.5771x over previous
import functools

import jax
import jax.numpy as jnp
from jax import lax
from jax.experimental import pallas as pl
from jax.experimental.pallas import tpu as pltpu
from jax.experimental.pallas import tpu_sc as plsc

B, C, N, K, COUT = 2, 128, 10000, 16, 128
NB = 1000
NW = 32
NPW = (B * N) // NW
G = 5
EPG = G * K
STEPS = NPW // G
NBUF = 4
LANES = 16


def _tc_tables_body(x_ref, w_ref, b_ref, y1_ref, y2_ref):
    xb = x_ref[0]
    w2 = w_ref[:, C:]
    a = w_ref[:, :C] - w2
    dn = (((0,), (1,)), ((), ()))
    y1_ref[...] = lax.dot_general(xb, a, dn, preferred_element_type=jnp.float32)
    y2_ref[...] = (lax.dot_general(xb, w2, dn, preferred_element_type=jnp.float32)
                   + b_ref[...])


_tc_tables = pl.pallas_call(
    _tc_tables_body,
    grid=(B,),
    in_specs=[
        pl.BlockSpec((1, C, N), lambda b: (b, 0, 0)),
        pl.BlockSpec((COUT, 2 * C), lambda b: (0, 0)),
        pl.BlockSpec((1, COUT), lambda b: (0, 0)),
    ],
    out_specs=[
        pl.BlockSpec((N, COUT), lambda b: (b, 0)),
        pl.BlockSpec((N, COUT), lambda b: (b, 0)),
    ],
    out_shape=[
        jax.ShapeDtypeStruct((B * N, COUT), jnp.float32),
        jax.ShapeDtypeStruct((B * N, COUT), jnp.float32),
    ],
)


@functools.partial(
    pl.kernel,
    mesh=plsc.VectorSubcoreMesh(core_axis_name="c", subcore_axis_name="s"),
    out_type=jax.ShapeDtypeStruct((B * N * COUT,), jnp.float32),
    scratch_types=(
        [pltpu.VMEM((NPW * K,), jnp.int32)] * 2
        + [pltpu.VMEM((EPG, COUT), jnp.float32) for _ in range(NBUF)]
        + [pltpu.VMEM((G * COUT,), jnp.float32) for _ in range(NBUF)]
        + [pltpu.SemaphoreType.DMA] * (3 * NBUF)
    ),
)
def _sc_edge_max(y1_hbm, y2_hbm, i1_hbm, i0_hbm, out_hbm, *refs):
    i1v, i0v = refs[0], refs[1]
    rs = refs[2:2 + NBUF]
    outs = refs[2 + NBUF:2 + 2 * NBUF]
    sems = refs[2 + 2 * NBUF:]
    s1 = sems[0:NBUF]
    s2 = sems[NBUF:2 * NBUF]
    so = sems[2 * NBUF:3 * NBUF]

    wid = lax.axis_index("s") * 2 + lax.axis_index("c")
    node_base = wid * NPW
    edge_base = node_base * K

    def issue1(s, b):
        pltpu.async_copy(y1_hbm.at[i1v.at[pl.ds(s * EPG, EPG)]], rs[b], s1[b])

    def wait1(b):
        pltpu.make_async_copy(y1_hbm.at[i1v.at[pl.ds(0, EPG)]], rs[b], s1[b]).wait()

    def issue2(s, b):
        pltpu.async_copy(y2_hbm.at[i0v.at[pl.ds(s * EPG, EPG)]], rs[b], s2[b],
                         add=True)

    def wait2(b):
        pltpu.make_async_copy(y2_hbm.at[i0v.at[pl.ds(0, EPG)]], rs[b], s2[b]).wait()

    def drain_store(b):
        pltpu.make_async_copy(outs[b], out_hbm.at[pl.ds(0, G * COUT)], so[b]).wait()

    def compute_store(s, b):
        r, outv = rs[b], outs[b]

        def gbody(g, carry):
            row = g * K
            for cb in range(COUT // LANES):
                co = cb * LANES

                def ld(k):
                    return r[row + k, pl.ds(co, LANES)]

                acc = None
                for k0 in range(0, K, 4):
                    m = jnp.maximum(jnp.maximum(ld(k0), ld(k0 + 1)),
                                    jnp.maximum(ld(k0 + 2), ld(k0 + 3)))
                    acc = m if acc is None else jnp.maximum(acc, m)
                outv[pl.ds(g * COUT + co, LANES)] = jnp.maximum(acc, 0.0)
            return carry

        lax.fori_loop(0, G, gbody, 0)
        pltpu.async_copy(
            outv, out_hbm.at[pl.ds((node_base + s * G) * COUT, G * COUT)], so[b])

    pltpu.sync_copy(i1_hbm.at[pl.ds(edge_base, NPW * K)], i1v)
    pltpu.sync_copy(i0_hbm.at[pl.ds(edge_base, NPW * K)], i0v)

    for b in range(NBUF):
        issue1(b, b)
    for b in range(2):
        wait1(b)
        issue2(b, b)

    def half(s, b, t):
        b2 = (b + 2) % NBUF

        @pl.when(s + 2 < STEPS)
        def _():
            wait1(b2)
            issue2(s + 2, b2)
        wait2(b)

        @pl.when(t > 0)
        def _():
            drain_store(b)
        compute_store(s, b)

        @pl.when(s + NBUF < STEPS)
        def _():
            issue1(s + NBUF, b)

    def body(t, carry):
        for b in range(NBUF):
            half(NBUF * t + b, b, t)
        return carry

    main_iters = STEPS // NBUF
    lax.fori_loop(0, main_iters, body, 0)

    for s in range(NBUF * main_iters, STEPS):
        b = s % NBUF
        wait2(b)
        drain_store(b)
        compute_store(s, b)
    for b in range(NBUF):
        drain_store(b)


def kernel(x, edge_index, W, b):
    y1, y2 = _tc_tables(x[..., 0], W, b.reshape(1, COUT))
    offs = (jnp.arange(B, dtype=jnp.int32) * N).reshape(B, 1, 1)
    i1f = (edge_index[1] + offs).reshape(-1)
    i0f = (edge_index[0] + offs).reshape(-1)
    out = _sc_edge_max(y1, y2, i1f, i0f)
    return out.reshape(B, N, COUT).transpose(0, 2, 1)[..., None]
